# Initial kernel scaffold; baseline (speedup 1.0000x reference)
import functools

import jax
import jax.numpy as jnp
from jax import lax
from jax.experimental import pallas as pl
from jax.experimental.pallas import tpu as pltpu
from jax.experimental.pallas import tpu_sc as plsc

N = 10000
E = 320000
H = 128
G = 64
NC = 2
NS = 16
NT = NC * NS
CH = 128
NCHUNK = -(-E // (NT * CH))
EPAD = NT * NCHUNK * CH
ACC_R = 10016
RPT = ACC_R // NS
TRASH = N

_mesh = plsc.VectorSubcoreMesh(core_axis_name="c", subcore_axis_name="s")


@functools.partial(
    pl.kernel,
    mesh=_mesh,
    out_type=jax.ShapeDtypeStruct((NC, ACC_R, 16), jnp.float32),
    scratch_types=[
        pltpu.VMEM((NCHUNK, CH), jnp.int32),
        pltpu.VMEM((CH, 16), jnp.float32),
        pltpu.VMEM_SHARED((ACC_R, 16), jnp.float32),
        pltpu.SemaphoreType.DMA,
    ],
)
def _sc_degree(dst_hbm, zero_hbm, ones_hbm, out_hbm, idx_v, ones_v, acc_sh, sem):
    c = lax.axis_index("c")
    s = lax.axis_index("s")
    wid = c * NS + s
    r0 = s * RPT
    pltpu.sync_copy(zero_hbm.at[pl.ds(r0, RPT)], acc_sh.at[pl.ds(r0, RPT)])
    pltpu.sync_copy(ones_hbm, ones_v)
    pltpu.sync_copy(dst_hbm.at[wid], idx_v)
    plsc.subcore_barrier()

    @pl.loop(0, NCHUNK)
    def _(i):
        pltpu.sync_copy(ones_v, acc_sh.at[idx_v.at[i]], add=True)

    plsc.subcore_barrier()
    pltpu.sync_copy(acc_sh.at[pl.ds(r0, RPT)], out_hbm.at[c, pl.ds(r0, RPT)])


@functools.partial(
    pl.kernel,
    mesh=_mesh,
    out_type=jax.ShapeDtypeStruct((NC, ACC_R, H), jnp.float32),
    scratch_types=[
        pltpu.VMEM((NCHUNK, CH), jnp.int32),
        pltpu.VMEM((NCHUNK, CH), jnp.int32),
        pltpu.VMEM((CH, H), jnp.float32),
        pltpu.VMEM_SHARED((ACC_R, H), jnp.float32),
        pltpu.SemaphoreType.DMA,
    ],
)
def _sc_msgpass(src_hbm, dst_hbm, hs_hbm, zero_hbm, out_hbm,
                sidx_v, didx_v, rows_v, acc_sh, sem):
    c = lax.axis_index("c")
    s = lax.axis_index("s")
    wid = c * NS + s
    r0 = s * RPT
    pltpu.sync_copy(zero_hbm.at[pl.ds(r0, RPT)], acc_sh.at[pl.ds(r0, RPT)])
    pltpu.sync_copy(src_hbm.at[wid], sidx_v)
    pltpu.sync_copy(dst_hbm.at[wid], didx_v)
    plsc.subcore_barrier()

    @pl.loop(0, NCHUNK)
    def _(i):
        pltpu.async_copy(hs_hbm.at[sidx_v.at[i]], rows_v, sem).wait()
        pltpu.sync_copy(rows_v, acc_sh.at[didx_v.at[i]], add=True)

    plsc.subcore_barrier()
    pltpu.sync_copy(acc_sh.at[pl.ds(r0, RPT)], out_hbm.at[c, pl.ds(r0, RPT)])


def _tc_matmul(x_ref, w_ref, o_ref):
    o_ref[...] = jnp.dot(x_ref[...], w_ref[...],
                         preferred_element_type=jnp.float32,
                         precision=lax.Precision.HIGHEST)


def _tc_scale(degp_ref, h1_ref, hs_ref, dis_ref):
    deg = degp_ref[0, :N, 0:1] + degp_ref[1, :N, 0:1] + 1.0
    dis = lax.rsqrt(deg)
    dis_ref[...] = dis
    hs_ref[...] = h1_ref[...] * dis


def _tc_mid(p_ref, hs_ref, dis_ref, b_ref, w_ref, o_ref):
    t = p_ref[0, :N, :] + p_ref[1, :N, :] + hs_ref[...]
    h = jnp.maximum(t * dis_ref[...] + b_ref[...], 0.0)
    o_ref[...] = jnp.dot(h, w_ref[...],
                         preferred_element_type=jnp.float32,
                         precision=lax.Precision.HIGHEST) * dis_ref[...]


def _tc_final(p_ref, hs_ref, dis_ref, b_ref, batch_ref, o_ref):
    t = p_ref[0, :N, :] + p_ref[1, :N, :] + hs_ref[...]
    h = jnp.maximum(t * dis_ref[...] + b_ref[...], 0.0)
    gid = lax.broadcasted_iota(jnp.int32, (G, N), 0)
    oh = (batch_ref[...] == gid).astype(jnp.float32)
    sums = jnp.dot(oh, h, preferred_element_type=jnp.float32,
                   precision=lax.Precision.HIGHEST)
    counts = jnp.sum(oh, axis=1, keepdims=True)
    o_ref[...] = sums / jnp.maximum(counts, 1.0)


def kernel(x, edge_index, batch, W1, b1, W2, b2):
    src = edge_index[0]
    dst = edge_index[1]
    pad = EPAD - E
    srcp = jnp.concatenate([src, jnp.zeros((pad,), jnp.int32)]).reshape(NT, NCHUNK, CH)
    dstp = jnp.concatenate([dst, jnp.full((pad,), TRASH, jnp.int32)]).reshape(NT, NCHUNK, CH)
    zeros_h = jnp.zeros((ACC_R, H), jnp.float32)
    zeros_16 = jnp.zeros((ACC_R, 16), jnp.float32)
    ones_16 = jnp.ones((CH, 16), jnp.float32)
    batch2 = batch.reshape(1, N)

    f32 = jnp.float32
    degp = _sc_degree(dstp, zeros_16, ones_16)
    h1 = pl.pallas_call(
        _tc_matmul, out_shape=jax.ShapeDtypeStruct((N, H), f32))(x, W1)

    hs1, dis = pl.pallas_call(
        _tc_scale,
        out_shape=(jax.ShapeDtypeStruct((N, H), f32),
                   jax.ShapeDtypeStruct((N, 1), f32)))(degp, h1)

    p1 = _sc_msgpass(srcp, dstp, hs1, zeros_h)

    hs2 = pl.pallas_call(
        _tc_mid, out_shape=jax.ShapeDtypeStruct((N, H), f32))(p1, hs1, dis, b1, W2)

    p2 = _sc_msgpass(srcp, dstp, hs2, zeros_h)

    out = pl.pallas_call(
        _tc_final, out_shape=jax.ShapeDtypeStruct((G, H), f32))(p2, hs2, dis, b2, batch2)
    return out


# speedup vs baseline: 11.8783x; 11.8783x over previous
import functools

import jax
import jax.numpy as jnp
from jax import lax
from jax.experimental import pallas as pl
from jax.experimental.pallas import tpu as pltpu
from jax.experimental.pallas import tpu_sc as plsc

N = 10000
E = 320000
H = 128
G = 64
NC = 2
NS = 16
NT = NC * NS
CH = 128
NCHUNK = -(-E // (NT * CH))
EPAD = NT * NCHUNK * CH
ACC_R = 10112
RPT = ACC_R // NS
TRASH = N

_mesh = plsc.VectorSubcoreMesh(core_axis_name="c", subcore_axis_name="s")


@functools.partial(
    pl.kernel,
    mesh=_mesh,
    out_type=jax.ShapeDtypeStruct((NC, ACC_R, H), jnp.float32),
    scratch_types=[
        pltpu.VMEM((NCHUNK, CH), jnp.int32),
        pltpu.VMEM((CH, H), jnp.float32),
        pltpu.VMEM_SHARED((ACC_R, H), jnp.float32),
        pltpu.SemaphoreType.DMA,
    ],
)
def _sc_degree(dst_hbm, zero_hbm, ones_hbm, out_hbm, idx_v, ones_v, acc_sh, sem):
    c = lax.axis_index("c")
    s = lax.axis_index("s")
    wid = c * NS + s
    r0 = pl.multiple_of(s * RPT, 8)
    pltpu.sync_copy(zero_hbm.at[pl.ds(r0, RPT)], acc_sh.at[pl.ds(r0, RPT)])
    pltpu.sync_copy(ones_hbm, ones_v)
    pltpu.sync_copy(dst_hbm.at[wid], idx_v)
    plsc.subcore_barrier()

    @pl.loop(0, NCHUNK)
    def _(i):
        pltpu.sync_copy(ones_v, acc_sh.at[idx_v.at[i]], add=True)

    plsc.subcore_barrier()
    pltpu.sync_copy(acc_sh.at[pl.ds(r0, RPT)], out_hbm.at[c, pl.ds(r0, RPT)])


@functools.partial(
    pl.kernel,
    mesh=_mesh,
    out_type=jax.ShapeDtypeStruct((NC, ACC_R, H), jnp.float32),
    scratch_types=[
        pltpu.VMEM((NCHUNK, CH), jnp.int32),
        pltpu.VMEM((NCHUNK, CH), jnp.int32),
        pltpu.VMEM((CH, H), jnp.float32),
        pltpu.VMEM_SHARED((ACC_R, H), jnp.float32),
        pltpu.SemaphoreType.DMA,
    ],
)
def _sc_msgpass(src_hbm, dst_hbm, hs_hbm, zero_hbm, out_hbm,
                sidx_v, didx_v, rows_v, acc_sh, sem):
    c = lax.axis_index("c")
    s = lax.axis_index("s")
    wid = c * NS + s
    r0 = pl.multiple_of(s * RPT, 8)
    pltpu.sync_copy(zero_hbm.at[pl.ds(r0, RPT)], acc_sh.at[pl.ds(r0, RPT)])
    pltpu.sync_copy(src_hbm.at[wid], sidx_v)
    pltpu.sync_copy(dst_hbm.at[wid], didx_v)
    plsc.subcore_barrier()

    @pl.loop(0, NCHUNK)
    def _(i):
        pltpu.async_copy(hs_hbm.at[sidx_v.at[i]], rows_v, sem).wait()
        pltpu.sync_copy(rows_v, acc_sh.at[didx_v.at[i]], add=True)

    plsc.subcore_barrier()
    pltpu.sync_copy(acc_sh.at[pl.ds(r0, RPT)], out_hbm.at[c, pl.ds(r0, RPT)])


def _tc_matmul(x_ref, w_ref, o_ref):
    o_ref[...] = jnp.dot(x_ref[...], w_ref[...],
                         preferred_element_type=jnp.float32,
                         precision=lax.Precision.HIGHEST)


def _tc_scale(degp_ref, h1_ref, hs_ref, dis_ref):
    deg = degp_ref[0, :N, 0:1] + degp_ref[1, :N, 0:1] + 1.0
    dis = lax.rsqrt(deg)
    dis_ref[...] = dis
    hs_ref[...] = h1_ref[...] * dis


def _tc_mid(p_ref, hs_ref, dis_ref, b_ref, w_ref, o_ref):
    t = p_ref[0, :N, :] + p_ref[1, :N, :] + hs_ref[...]
    h = jnp.maximum(t * dis_ref[...] + b_ref[...], 0.0)
    o_ref[...] = jnp.dot(h, w_ref[...],
                         preferred_element_type=jnp.float32,
                         precision=lax.Precision.HIGHEST) * dis_ref[...]


def _tc_final(p_ref, hs_ref, dis_ref, b_ref, batch_ref, o_ref):
    t = p_ref[0, :N, :] + p_ref[1, :N, :] + hs_ref[...]
    h = jnp.maximum(t * dis_ref[...] + b_ref[...], 0.0)
    gid = lax.broadcasted_iota(jnp.int32, (G, N), 0)
    oh = (batch_ref[...] == gid).astype(jnp.float32)
    sums = jnp.dot(oh, h, preferred_element_type=jnp.float32,
                   precision=lax.Precision.HIGHEST)
    counts = jnp.sum(oh, axis=1, keepdims=True)
    o_ref[...] = sums / jnp.maximum(counts, 1.0)


def kernel(x, edge_index, batch, W1, b1, W2, b2):
    src = edge_index[0]
    dst = edge_index[1]
    pad = EPAD - E
    srcp = jnp.concatenate([src, jnp.zeros((pad,), jnp.int32)]).reshape(NT, NCHUNK, CH)
    dstp = jnp.concatenate([dst, jnp.full((pad,), TRASH, jnp.int32)]).reshape(NT, NCHUNK, CH)
    zeros_h = jnp.zeros((ACC_R, H), jnp.float32)
    ones_h = jnp.ones((CH, H), jnp.float32)
    batch2 = batch.reshape(1, N)

    f32 = jnp.float32
    degp = _sc_degree(dstp, zeros_h, ones_h)
    h1 = pl.pallas_call(
        _tc_matmul, out_shape=jax.ShapeDtypeStruct((N, H), f32))(x, W1)

    hs1, dis = pl.pallas_call(
        _tc_scale,
        out_shape=(jax.ShapeDtypeStruct((N, H), f32),
                   jax.ShapeDtypeStruct((N, 1), f32)))(degp, h1)

    p1 = _sc_msgpass(srcp, dstp, hs1, zeros_h)

    hs2 = pl.pallas_call(
        _tc_mid, out_shape=jax.ShapeDtypeStruct((N, H), f32))(p1, hs1, dis, b1, W2)

    p2 = _sc_msgpass(srcp, dstp, hs2, zeros_h)

    out = pl.pallas_call(
        _tc_final, out_shape=jax.ShapeDtypeStruct((G, H), f32))(p2, hs2, dis, b2, batch2)
    return out


# speedup vs baseline: 14.3663x; 1.2095x over previous
import functools

import jax
import jax.numpy as jnp
from jax import lax
from jax.experimental import pallas as pl
from jax.experimental.pallas import tpu as pltpu
from jax.experimental.pallas import tpu_sc as plsc

N = 10000
E = 320000
H = 128
G = 64
NC = 2
NS = 16
NT = NC * NS
CH = 128
NCHUNK = -(-E // (NT * CH))
EPAD = NT * NCHUNK * CH
ACC_R = 10112
RPT = ACC_R // NS
TRASH = N

_mesh = plsc.VectorSubcoreMesh(core_axis_name="c", subcore_axis_name="s")


@functools.partial(
    pl.kernel,
    mesh=_mesh,
    out_type=jax.ShapeDtypeStruct((NC, ACC_R, H), jnp.float32),
    scratch_types=[
        pltpu.VMEM((NCHUNK, CH), jnp.int32),
        pltpu.VMEM((CH, H), jnp.float32),
        pltpu.VMEM_SHARED((ACC_R, H), jnp.float32),
        pltpu.SemaphoreType.DMA,
    ],
)
def _sc_degree(dst_hbm, zero_hbm, ones_hbm, out_hbm, idx_v, ones_v, acc_sh, sem):
    c = lax.axis_index("c")
    s = lax.axis_index("s")
    wid = c * NS + s
    r0 = pl.multiple_of(s * RPT, 8)
    pltpu.sync_copy(zero_hbm.at[pl.ds(r0, RPT)], acc_sh.at[pl.ds(r0, RPT)])
    pltpu.sync_copy(ones_hbm, ones_v)
    pltpu.sync_copy(dst_hbm.at[wid], idx_v)
    plsc.subcore_barrier()

    @pl.loop(0, NCHUNK)
    def _(i):
        pltpu.sync_copy(ones_v, acc_sh.at[idx_v.at[i]], add=True)

    plsc.subcore_barrier()
    pltpu.sync_copy(acc_sh.at[pl.ds(r0, RPT)], out_hbm.at[c, pl.ds(r0, RPT)])


@functools.partial(
    pl.kernel,
    mesh=_mesh,
    out_type=jax.ShapeDtypeStruct((NC, ACC_R, H), jnp.float32),
    scratch_types=[
        pltpu.VMEM((2, CH), jnp.int32),
        pltpu.VMEM((2, CH), jnp.int32),
        pltpu.VMEM((CH, H), jnp.float32),
        pltpu.VMEM((CH, H), jnp.float32),
        pltpu.VMEM_SHARED((ACC_R, H), jnp.float32),
        pltpu.SemaphoreType.DMA,
        pltpu.SemaphoreType.DMA,
    ],
)
def _sc_msgpass(idx_hbm, hs_hbm, zero_hbm, out_hbm,
                idx_a, idx_b, rows_a, rows_b, acc_sh, sem_a, sem_b):
    c = lax.axis_index("c")
    s = lax.axis_index("s")
    wid = c * NS + s
    r0 = pl.multiple_of(s * RPT, 8)
    pltpu.sync_copy(zero_hbm.at[pl.ds(r0, RPT)], acc_sh.at[pl.ds(r0, RPT)])
    plsc.subcore_barrier()

    pltpu.sync_copy(idx_hbm.at[wid, 0], idx_a)
    pltpu.async_copy(hs_hbm.at[idx_a.at[0]], rows_a, sem_a)
    pltpu.sync_copy(idx_hbm.at[wid, 1], idx_b)

    @pl.loop(0, (NCHUNK - 1) // 2)
    def _(j):
        i = j * 2
        pltpu.async_copy(hs_hbm.at[idx_b.at[0]], rows_b, sem_b)
        pltpu.make_async_copy(hs_hbm.at[idx_a.at[0]], rows_a, sem_a).wait()
        pltpu.sync_copy(rows_a, acc_sh.at[idx_a.at[1]], add=True)
        pltpu.sync_copy(idx_hbm.at[wid, i + 2], idx_a)
        pltpu.async_copy(hs_hbm.at[idx_a.at[0]], rows_a, sem_a)
        pltpu.make_async_copy(hs_hbm.at[idx_b.at[0]], rows_b, sem_b).wait()
        pltpu.sync_copy(rows_b, acc_sh.at[idx_b.at[1]], add=True)
        pltpu.sync_copy(idx_hbm.at[wid, jnp.minimum(i + 3, NCHUNK - 1)], idx_b)

    pltpu.make_async_copy(hs_hbm.at[idx_a.at[0]], rows_a, sem_a).wait()
    pltpu.sync_copy(rows_a, acc_sh.at[idx_a.at[1]], add=True)

    plsc.subcore_barrier()
    pltpu.sync_copy(acc_sh.at[pl.ds(r0, RPT)], out_hbm.at[c, pl.ds(r0, RPT)])


def _tc_matmul(x_ref, w_ref, o_ref):
    o_ref[...] = jnp.dot(x_ref[...], w_ref[...],
                         preferred_element_type=jnp.float32,
                         precision=lax.Precision.HIGHEST)


def _tc_scale(degp_ref, h1_ref, hs_ref, dis_ref):
    deg = degp_ref[0, :N, 0:1] + degp_ref[1, :N, 0:1] + 1.0
    dis = lax.rsqrt(deg)
    dis_ref[...] = dis
    hs_ref[...] = h1_ref[...] * dis


def _tc_mid(p_ref, hs_ref, dis_ref, b_ref, w_ref, o_ref):
    t = p_ref[0, :N, :] + p_ref[1, :N, :] + hs_ref[...]
    h = jnp.maximum(t * dis_ref[...] + b_ref[...], 0.0)
    o_ref[...] = jnp.dot(h, w_ref[...],
                         preferred_element_type=jnp.float32,
                         precision=lax.Precision.HIGHEST) * dis_ref[...]


def _tc_final(p_ref, hs_ref, dis_ref, b_ref, batch_ref, o_ref):
    t = p_ref[0, :N, :] + p_ref[1, :N, :] + hs_ref[...]
    h = jnp.maximum(t * dis_ref[...] + b_ref[...], 0.0)
    gid = lax.broadcasted_iota(jnp.int32, (G, N), 0)
    oh = (batch_ref[...] == gid).astype(jnp.float32)
    sums = jnp.dot(oh, h, preferred_element_type=jnp.float32,
                   precision=lax.Precision.HIGHEST)
    counts = jnp.sum(oh, axis=1, keepdims=True)
    o_ref[...] = sums / jnp.maximum(counts, 1.0)


def kernel(x, edge_index, batch, W1, b1, W2, b2):
    src = edge_index[0]
    dst = edge_index[1]
    pad = EPAD - E
    srcp = jnp.concatenate([src, jnp.zeros((pad,), jnp.int32)]).reshape(NT, NCHUNK, CH)
    dstp = jnp.concatenate([dst, jnp.full((pad,), TRASH, jnp.int32)]).reshape(NT, NCHUNK, CH)
    idx2 = jnp.stack([srcp, dstp], axis=2)
    zeros_h = jnp.zeros((ACC_R, H), jnp.float32)
    ones_h = jnp.ones((CH, H), jnp.float32)
    batch2 = batch.reshape(1, N)

    f32 = jnp.float32
    degp = _sc_degree(dstp, zeros_h, ones_h)
    h1 = pl.pallas_call(
        _tc_matmul, out_shape=jax.ShapeDtypeStruct((N, H), f32))(x, W1)

    hs1, dis = pl.pallas_call(
        _tc_scale,
        out_shape=(jax.ShapeDtypeStruct((N, H), f32),
                   jax.ShapeDtypeStruct((N, 1), f32)))(degp, h1)

    p1 = _sc_msgpass(idx2, hs1, zeros_h)

    hs2 = pl.pallas_call(
        _tc_mid, out_shape=jax.ShapeDtypeStruct((N, H), f32))(p1, hs1, dis, b1, W2)

    p2 = _sc_msgpass(idx2, hs2, zeros_h)

    out = pl.pallas_call(
        _tc_final, out_shape=jax.ShapeDtypeStruct((G, H), f32))(p2, hs2, dis, b2, batch2)
    return out


# speedup vs baseline: 15.8862x; 1.1058x over previous
import functools

import jax
import jax.numpy as jnp
from jax import lax
from jax.experimental import pallas as pl
from jax.experimental.pallas import tpu as pltpu
from jax.experimental.pallas import tpu_sc as plsc

N = 10000
E = 320000
H = 128
G = 64
NC = 2
NS = 16
NT = NC * NS
CH = 128
NCHUNK = -(-E // (NT * CH))
EPAD = NT * NCHUNK * CH
NCH0 = 109
NCH1 = 49
EPAD2 = NS * (NCH0 + NCH1) * CH
ACC_R = 10112
RPT = ACC_R // NS
TRASH = N

_mesh = plsc.VectorSubcoreMesh(core_axis_name="c", subcore_axis_name="s")


@functools.partial(
    pl.kernel,
    mesh=_mesh,
    out_type=jax.ShapeDtypeStruct((NC, ACC_R, H), jnp.float32),
    scratch_types=[
        pltpu.VMEM((NCHUNK, CH), jnp.int32),
        pltpu.VMEM((CH, H), jnp.float32),
        pltpu.VMEM_SHARED((ACC_R, H), jnp.float32),
        pltpu.SemaphoreType.DMA,
    ],
)
def _sc_degree(dst_hbm, zero_hbm, ones_hbm, out_hbm, idx_v, ones_v, acc_sh, sem):
    c = lax.axis_index("c")
    s = lax.axis_index("s")
    wid = c * NS + s
    r0 = pl.multiple_of(s * RPT, 8)
    pltpu.sync_copy(zero_hbm.at[pl.ds(r0, RPT)], acc_sh.at[pl.ds(r0, RPT)])
    pltpu.sync_copy(ones_hbm, ones_v)
    pltpu.sync_copy(dst_hbm.at[wid], idx_v)
    plsc.subcore_barrier()

    @pl.loop(0, NCHUNK)
    def _(i):
        pltpu.sync_copy(ones_v, acc_sh.at[idx_v.at[i]], add=True)

    plsc.subcore_barrier()
    pltpu.sync_copy(acc_sh.at[pl.ds(r0, RPT)], out_hbm.at[c, pl.ds(r0, RPT)])


@functools.partial(
    pl.kernel,
    mesh=_mesh,
    out_type=jax.ShapeDtypeStruct((NC, ACC_R, H), jnp.float32),
    scratch_types=[
        pltpu.VMEM((2, CH), jnp.int32),
        pltpu.VMEM((2, CH), jnp.int32),
        pltpu.VMEM((CH, H), jnp.float32),
        pltpu.VMEM((CH, H), jnp.float32),
        pltpu.VMEM_SHARED((ACC_R, H), jnp.float32),
        pltpu.SemaphoreType.DMA,
        pltpu.SemaphoreType.DMA,
    ],
)
def _sc_msgpass(idx0_hbm, idx1_hbm, hs_hbm, zero_hbm, out_hbm,
                idx_a, idx_b, rows_a, rows_b, acc_sh, sem_a, sem_b):
    c = lax.axis_index("c")
    s = lax.axis_index("s")
    r0 = pl.multiple_of(s * RPT, 8)
    pltpu.sync_copy(zero_hbm.at[pl.ds(r0, RPT)], acc_sh.at[pl.ds(r0, RPT)])
    plsc.subcore_barrier()

    def _pipe(idx_hbm, nch):
        pltpu.sync_copy(idx_hbm.at[s, 0], idx_a)
        pltpu.async_copy(hs_hbm.at[idx_a.at[0]], rows_a, sem_a)
        pltpu.sync_copy(idx_hbm.at[s, 1], idx_b)

        @pl.loop(0, (nch - 1) // 2)
        def _(j):
            i = j * 2
            pltpu.async_copy(hs_hbm.at[idx_b.at[0]], rows_b, sem_b)
            pltpu.make_async_copy(hs_hbm.at[idx_a.at[0]], rows_a, sem_a).wait()
            pltpu.sync_copy(rows_a, acc_sh.at[idx_a.at[1]], add=True)
            pltpu.sync_copy(idx_hbm.at[s, i + 2], idx_a)
            pltpu.async_copy(hs_hbm.at[idx_a.at[0]], rows_a, sem_a)
            pltpu.make_async_copy(hs_hbm.at[idx_b.at[0]], rows_b, sem_b).wait()
            pltpu.sync_copy(rows_b, acc_sh.at[idx_b.at[1]], add=True)
            pltpu.sync_copy(idx_hbm.at[s, jnp.minimum(i + 3, nch - 1)], idx_b)

        pltpu.make_async_copy(hs_hbm.at[idx_a.at[0]], rows_a, sem_a).wait()
        pltpu.sync_copy(rows_a, acc_sh.at[idx_a.at[1]], add=True)

    @pl.when(c == 0)
    def _():
        _pipe(idx0_hbm, NCH0)

    @pl.when(c == 1)
    def _():
        _pipe(idx1_hbm, NCH1)

    plsc.subcore_barrier()
    pltpu.sync_copy(acc_sh.at[pl.ds(r0, RPT)], out_hbm.at[c, pl.ds(r0, RPT)])


def _tc_matmul(x_ref, w_ref, o_ref):
    o_ref[...] = jnp.dot(x_ref[...], w_ref[...],
                         preferred_element_type=jnp.float32,
                         precision=lax.Precision.HIGHEST)


def _tc_scale(degp_ref, h1_ref, hs_ref, dis_ref):
    deg = degp_ref[0, :N, 0:1] + degp_ref[1, :N, 0:1] + 1.0
    dis = lax.rsqrt(deg)
    dis_ref[...] = dis
    hs_ref[...] = h1_ref[...] * dis


def _tc_mid(p_ref, hs_ref, dis_ref, b_ref, w_ref, o_ref):
    t = p_ref[0, :N, :] + p_ref[1, :N, :] + hs_ref[...]
    h = jnp.maximum(t * dis_ref[...] + b_ref[...], 0.0)
    o_ref[...] = jnp.dot(h, w_ref[...],
                         preferred_element_type=jnp.float32,
                         precision=lax.Precision.HIGHEST) * dis_ref[...]


def _tc_final(p_ref, hs_ref, dis_ref, b_ref, batch_ref, o_ref):
    t = p_ref[0, :N, :] + p_ref[1, :N, :] + hs_ref[...]
    h = jnp.maximum(t * dis_ref[...] + b_ref[...], 0.0)
    gid = lax.broadcasted_iota(jnp.int32, (G, N), 0)
    oh = (batch_ref[...] == gid).astype(jnp.float32)
    sums = jnp.dot(oh, h, preferred_element_type=jnp.float32,
                   precision=lax.Precision.HIGHEST)
    counts = jnp.sum(oh, axis=1, keepdims=True)
    o_ref[...] = sums / jnp.maximum(counts, 1.0)


def kernel(x, edge_index, batch, W1, b1, W2, b2):
    src = edge_index[0]
    dst = edge_index[1]
    pad = EPAD - E
    dstp = jnp.concatenate([dst, jnp.full((pad,), TRASH, jnp.int32)]).reshape(NT, NCHUNK, CH)
    pad2 = EPAD2 - E
    srcf = jnp.concatenate([src, jnp.zeros((pad2,), jnp.int32)])
    dstf = jnp.concatenate([dst, jnp.full((pad2,), TRASH, jnp.int32)])
    n0 = NS * NCH0 * CH
    idx0 = jnp.stack([srcf[:n0].reshape(NS, NCH0, CH),
                      dstf[:n0].reshape(NS, NCH0, CH)], axis=2)
    idx1 = jnp.stack([srcf[n0:].reshape(NS, NCH1, CH),
                      dstf[n0:].reshape(NS, NCH1, CH)], axis=2)
    zeros_h = jnp.zeros((ACC_R, H), jnp.float32)
    ones_h = jnp.ones((CH, H), jnp.float32)
    batch2 = batch.reshape(1, N)

    f32 = jnp.float32
    degp = _sc_degree(dstp, zeros_h, ones_h)
    h1 = pl.pallas_call(
        _tc_matmul, out_shape=jax.ShapeDtypeStruct((N, H), f32))(x, W1)

    hs1, dis = pl.pallas_call(
        _tc_scale,
        out_shape=(jax.ShapeDtypeStruct((N, H), f32),
                   jax.ShapeDtypeStruct((N, 1), f32)))(degp, h1)

    p1 = _sc_msgpass(idx0, idx1, hs1, zeros_h)

    hs2 = pl.pallas_call(
        _tc_mid, out_shape=jax.ShapeDtypeStruct((N, H), f32))(p1, hs1, dis, b1, W2)

    p2 = _sc_msgpass(idx0, idx1, hs2, zeros_h)

    out = pl.pallas_call(
        _tc_final, out_shape=jax.ShapeDtypeStruct((G, H), f32))(p2, hs2, dis, b2, batch2)
    return out
